# Initial kernel scaffold; baseline (speedup 1.0000x reference)
import jax
import jax.numpy as jnp
from jax.experimental import pallas as pl


def kernel(h, edge_index, W1, b1, W2, b2):
    raise NotImplementedError("write your pallas kernel here")


# speedup vs baseline: 8.6847x; 8.6847x over previous
import jax
import jax.numpy as jnp
from jax import lax
from jax.experimental import pallas as pl
from jax.experimental.pallas import tpu as pltpu
from jax.experimental.pallas import tpu_sc as plsc

NN = 10000
EE = 320000
FIN = 128
FOUT = 256
NCLS = 64

NC, NS = 2, 16
K = 80
ROWS_ALL = EE // K
ROWS_TILE_A = ROWS_ALL // NS
ROWS_TILE_C = ROWS_ALL // (NC * NS)
RPT = NN // NS

def _deg_body(src3_hbm, dst3_hbm, z1_hbm, deg_hbm, idx_v, ones_v, tb_v, acc_s):
    c = lax.axis_index("c")
    s = lax.axis_index("s")

    @pl.when(s < 10)
    def _():
        pltpu.sync_copy(z1_hbm, tb_v)
        pltpu.sync_copy(tb_v, acc_s.at[pl.ds(s * 1000, 1000)])

    for u in range(K // 16):
        ones_v[pl.ds(u * 16, 16)] = jnp.full((16,), 1.0, jnp.float32)

    @pl.when(c == 0)
    def _():
        pltpu.sync_copy(src3_hbm.at[2 * s], idx_v.at[pl.ds(0, ROWS_TILE_C)])
        pltpu.sync_copy(src3_hbm.at[2 * s + 1],
                        idx_v.at[pl.ds(ROWS_TILE_C, ROWS_TILE_C)])

    @pl.when(c == 1)
    def _():
        pltpu.sync_copy(dst3_hbm.at[2 * s], idx_v.at[pl.ds(0, ROWS_TILE_C)])
        pltpu.sync_copy(dst3_hbm.at[2 * s + 1],
                        idx_v.at[pl.ds(ROWS_TILE_C, ROWS_TILE_C)])

    plsc.subcore_barrier()

    def chunk(j, carry):
        pltpu.sync_copy(ones_v, acc_s.at[idx_v.at[j]], add=True)
        return carry

    lax.fori_loop(0, ROWS_TILE_A, chunk, None)
    plsc.subcore_barrier()

    @pl.when(s < 10)
    def _():
        pltpu.sync_copy(acc_s.at[pl.ds(s * 1000, 1000)], tb_v)
        pltpu.sync_copy(tb_v, deg_hbm.at[pl.ds(c * NN + s * 1000, 1000)])


_sc_calls = {}


def _get_sc_calls():
    if _sc_calls:
        return _sc_calls
    mesh = plsc.VectorSubcoreMesh(
        core_axis_name="c", subcore_axis_name="s",
        num_cores=NC, num_subcores=NS)
    _sc_calls["deg"] = pl.kernel(
        _deg_body,
        out_type=jax.ShapeDtypeStruct((2 * NN,), jnp.float32),
        mesh=mesh,
        scratch_types=[
            pltpu.VMEM((ROWS_TILE_A, K), jnp.int32),
            pltpu.VMEM((K,), jnp.float32),
            pltpu.VMEM((1000,), jnp.float32),
            pltpu.VMEM_SHARED((NN,), jnp.float32),
        ],
    )
    _sc_calls["agg"] = pl.kernel(
        _agg_body,
        out_type=(jax.ShapeDtypeStruct((2 * NN, FIN), jnp.float32),
                  jax.ShapeDtypeStruct((2 * NN,), jnp.float32)),
        mesh=mesh,
        scratch_types=[
            pltpu.VMEM((ROWS_TILE_C, K), jnp.int32),
            pltpu.VMEM((ROWS_TILE_C, K), jnp.int32),
            pltpu.VMEM((K, FIN), jnp.float32),
            pltpu.VMEM((K,), jnp.float32),
            pltpu.VMEM((BR, FIN), jnp.float32),
            pltpu.VMEM((1000,), jnp.float32),
            pltpu.VMEM_SHARED((NN, FIN), jnp.float32),
            pltpu.VMEM_SHARED((NN,), jnp.float32),
        ],
    )
    return _sc_calls


BR = 48
NBC = 13
TAIL = NN - NS * NBC * BR


def _agg_body(src3_hbm, dst3_hbm, hs_hbm, rin_hbm, z2_hbm, z1_hbm,
              agg_hbm, t_hbm, sidx_v, didx_v, rows_v, rv_v, zb_v, tb_v,
              acc_s, tacc_s):
    c = lax.axis_index("c")
    s = lax.axis_index("s")

    @pl.when(s < 10)
    def _():
        pltpu.sync_copy(z1_hbm, tb_v)
        pltpu.sync_copy(tb_v, tacc_s.at[pl.ds(s * 1000, 1000)])

    pltpu.sync_copy(z2_hbm, zb_v)
    base = s * NBC * BR
    for q in range(NBC):
        pltpu.sync_copy(zb_v, acc_s.at[pl.ds(base + q * BR, BR)])

    @pl.when(s == 0)
    def _():
        pltpu.sync_copy(zb_v.at[pl.ds(0, TAIL)], acc_s.at[pl.ds(NN - TAIL, TAIL)])

    w = c * NS + s
    pltpu.sync_copy(src3_hbm.at[w], sidx_v)
    pltpu.sync_copy(dst3_hbm.at[w], didx_v)
    plsc.subcore_barrier()

    def chunk(j, carry):
        pltpu.sync_copy(hs_hbm.at[sidx_v.at[j]], rows_v)
        pltpu.sync_copy(rows_v, acc_s.at[didx_v.at[j]], add=True)
        pltpu.sync_copy(rin_hbm.at[didx_v.at[j]], rv_v)
        pltpu.sync_copy(rv_v, tacc_s.at[sidx_v.at[j]], add=True)
        return carry

    lax.fori_loop(0, ROWS_TILE_C, chunk, None)
    plsc.subcore_barrier()

    for q in range(NBC):
        pltpu.sync_copy(acc_s.at[pl.ds(base + q * BR, BR)], zb_v)
        pltpu.sync_copy(zb_v, agg_hbm.at[pl.ds(c * NN + base + q * BR, BR)])

    @pl.when(s == 0)
    def _():
        pltpu.sync_copy(acc_s.at[pl.ds(NN - TAIL, TAIL)],
                        rows_v.at[pl.ds(0, TAIL)])
        pltpu.sync_copy(rows_v.at[pl.ds(0, TAIL)],
                        agg_hbm.at[pl.ds(c * NN + NN - TAIL, TAIL)])

    @pl.when(s < 10)
    def _():
        pltpu.sync_copy(tacc_s.at[pl.ds(s * 1000, 1000)], tb_v)
        pltpu.sync_copy(tb_v, t_hbm.at[pl.ds(c * NN + s * 1000, 1000)])


def _scale_body(h_ref, do_ref, di_ref, hs_ref, rin_ref, rout_ref):
    ro = lax.rsqrt(jnp.maximum(do_ref[...], 1.0))
    ri = lax.rsqrt(jnp.maximum(di_ref[...], 1.0))
    hs_ref[...] = h_ref[...] * ro
    rin_ref[...] = ri
    rout_ref[...] = ro


_BB = 2000
_scale_call = pl.pallas_call(
    _scale_body,
    grid=(NN // _BB,),
    in_specs=[
        pl.BlockSpec((_BB, FIN), lambda i: (i, 0)),
        pl.BlockSpec((_BB, 1), lambda i: (i, 0)),
        pl.BlockSpec((_BB, 1), lambda i: (i, 0)),
    ],
    out_specs=[
        pl.BlockSpec((_BB, FIN), lambda i: (i, 0)),
        pl.BlockSpec((_BB, 1), lambda i: (i, 0)),
        pl.BlockSpec((_BB, 1), lambda i: (i, 0)),
    ],
    out_shape=[
        jax.ShapeDtypeStruct((NN, FIN), jnp.float32),
        jax.ShapeDtypeStruct((NN, 1), jnp.float32),
        jax.ShapeDtypeStruct((NN, 1), jnp.float32),
    ],
)


def _out_body(p_ref, t_ref, rin_ref, rout_ref, w1_ref, b1_ref, w2_ref, b2_ref,
              o_ref, v_acc):
    i = pl.program_id(0)

    @pl.when(i == 0)
    def _():
        v_acc[...] = jnp.zeros_like(v_acc)

    aggp = p_ref[0] + p_ref[1]
    x1 = jnp.dot(aggp, w1_ref[...], preferred_element_type=jnp.float32)
    x1 = jnp.maximum(x1 * rin_ref[...] + b1_ref[...], 0.0)
    w = rout_ref[...] * (t_ref[0] + t_ref[1])
    v_acc[...] += lax.dot_general(w, x1, (((0,), (0,)), ((), ())),
                                  preferred_element_type=jnp.float32)

    @pl.when(i == pl.num_programs(0) - 1)
    def _():
        o_ref[...] = (jnp.dot(v_acc[...], w2_ref[...],
                              preferred_element_type=jnp.float32) * (1.0 / NN)
                      + b2_ref[...])


_BD = 1000
_out_call = pl.pallas_call(
    _out_body,
    grid=(NN // _BD,),
    in_specs=[
        pl.BlockSpec((2, _BD, FIN), lambda i: (0, i, 0)),
        pl.BlockSpec((2, _BD, 1), lambda i: (0, i, 0)),
        pl.BlockSpec((_BD, 1), lambda i: (i, 0)),
        pl.BlockSpec((_BD, 1), lambda i: (i, 0)),
        pl.BlockSpec((FIN, FOUT), lambda i: (0, 0)),
        pl.BlockSpec((1, FOUT), lambda i: (0, 0)),
        pl.BlockSpec((FOUT, NCLS), lambda i: (0, 0)),
        pl.BlockSpec((1, NCLS), lambda i: (0, 0)),
    ],
    out_specs=pl.BlockSpec((1, NCLS), lambda i: (0, 0)),
    out_shape=jax.ShapeDtypeStruct((1, NCLS), jnp.float32),
    scratch_shapes=[pltpu.VMEM((1, FOUT), jnp.float32)],
)


def kernel(h, edge_index, W1, b1, W2, b2):
    ei = edge_index.astype(jnp.int32)
    src2 = ei[0].reshape(NC * NS, ROWS_TILE_C, K)
    dst2 = ei[1].reshape(NC * NS, ROWS_TILE_C, K)
    z1 = jnp.zeros((1000,), jnp.float32)
    z2 = jnp.zeros((BR, FIN), jnp.float32)

    sc = _get_sc_calls()
    deg = sc["deg"](src2, dst2, z1)
    do = deg[:NN].reshape(NN, 1)
    di = deg[NN:].reshape(NN, 1)

    hs, rin, rout = _scale_call(h, do, di)

    aggp, tp = sc["agg"](src2, dst2, hs, rin.reshape(NN), z2, z1)

    return _out_call(aggp.reshape(2, NN, FIN), tp.reshape(2, NN, 1),
                     rin, rout, W1, b1.reshape(1, FOUT), W2, b2.reshape(1, NCLS))


# speedup vs baseline: 13.6391x; 1.5705x over previous
import jax
import jax.numpy as jnp
from jax import lax
from jax.experimental import pallas as pl
from jax.experimental.pallas import tpu as pltpu
from jax.experimental.pallas import tpu_sc as plsc

NN = 10000
EE = 320000
FIN = 128
FOUT = 256
NCLS = 64

NC, NS = 2, 16
K = 80
WIN = 25
ROWS_ALL = EE // K
ROWS_TILE_A = ROWS_ALL // NS
ROWS_TILE_C = ROWS_ALL // (NC * NS)
NWIN = ROWS_TILE_C // WIN
RPT = NN // NS

def _deg_body(src3_hbm, dst3_hbm, z1_hbm, deg_hbm, idx_v, ones_v, tb_v, acc_s,
              dsem):
    c = lax.axis_index("c")
    s = lax.axis_index("s")

    @pl.when(s < 10)
    def _():
        pltpu.sync_copy(z1_hbm, tb_v)
        pltpu.sync_copy(tb_v, acc_s.at[pl.ds(s * 1000, 1000)])

    for u in range(K // 16):
        ones_v[pl.ds(u * 16, 16)] = jnp.full((16,), 1.0, jnp.float32)

    @pl.when(c == 0)
    def _():
        pltpu.sync_copy(src3_hbm.at[2 * s], idx_v.at[0])
        pltpu.sync_copy(src3_hbm.at[2 * s + 1], idx_v.at[1])

    @pl.when(c == 1)
    def _():
        pltpu.sync_copy(dst3_hbm.at[2 * s], idx_v.at[0])
        pltpu.sync_copy(dst3_hbm.at[2 * s + 1], idx_v.at[1])

    plsc.subcore_barrier()

    KF = 8

    def row(j):
        jj = lax.rem(j, NWIN * WIN)
        return (idx_v.at[lax.div(j, NWIN * WIN)]
                .at[lax.div(jj, WIN)].at[lax.rem(jj, WIN)])

    def chunk(j, carry):
        pltpu.async_copy(ones_v, acc_s.at[row(j)], dsem, add=True)

        @pl.when(j >= KF)
        def _():
            pltpu.make_async_copy(ones_v, acc_s.at[row(j)], dsem).wait()

        return carry

    lax.fori_loop(0, ROWS_TILE_A, chunk, None)
    for j in range(KF):
        pltpu.make_async_copy(ones_v, acc_s.at[row(j)], dsem).wait()
    plsc.subcore_barrier()

    @pl.when(s < 10)
    def _():
        pltpu.sync_copy(acc_s.at[pl.ds(s * 1000, 1000)], tb_v)
        pltpu.sync_copy(tb_v, deg_hbm.at[pl.ds(c * NN + s * 1000, 1000)])


_sc_calls = {}


def _get_sc_calls():
    if _sc_calls:
        return _sc_calls
    mesh = plsc.VectorSubcoreMesh(
        core_axis_name="c", subcore_axis_name="s",
        num_cores=NC, num_subcores=NS)
    _sc_calls["deg"] = pl.kernel(
        _deg_body,
        out_type=jax.ShapeDtypeStruct((2 * NN,), jnp.float32),
        mesh=mesh,
        scratch_types=[
            pltpu.VMEM((2, NWIN, WIN, K), jnp.int32),
            pltpu.VMEM((K,), jnp.float32),
            pltpu.VMEM((1000,), jnp.float32),
            pltpu.VMEM_SHARED((NN,), jnp.float32),
            pltpu.SemaphoreType.DMA,
        ],
    )
    _sc_calls["agg"] = pl.kernel(
        _agg_body,
        out_type=(jax.ShapeDtypeStruct((2 * NN, FIN), jnp.float32),
                  jax.ShapeDtypeStruct((2 * NN,), jnp.float32)),
        mesh=mesh,
        scratch_types=[
            pltpu.VMEM((2, WIN, K), jnp.int32),
            pltpu.VMEM((2, WIN, K), jnp.int32),
            pltpu.VMEM((2, K, FIN), jnp.float32),
            pltpu.VMEM((2, K), jnp.float32),
            pltpu.VMEM((BR, FIN), jnp.float32),
            pltpu.VMEM((1000,), jnp.float32),
            pltpu.VMEM_SHARED((NN, FIN), jnp.float32),
            pltpu.VMEM_SHARED((NN,), jnp.float32),
            pltpu.SemaphoreType.DMA,
            pltpu.SemaphoreType.DMA,
            pltpu.SemaphoreType.DMA,
            pltpu.SemaphoreType.DMA,
            pltpu.SemaphoreType.DMA,
        ],
    )
    return _sc_calls


BR = 24
NBC = 26
TAIL = NN - NS * NBC * BR


def _agg_body(src3_hbm, dst3_hbm, hs_hbm, rin_hbm, z2_hbm, z1_hbm,
              agg_hbm, t_hbm, sidx_v, didx_v, rows_v, rv_v, zb_v, tb_v,
              acc_s, tacc_s, gsem, grsem, ssem, stsem, isem):
    c = lax.axis_index("c")
    s = lax.axis_index("s")

    @pl.when(s < 10)
    def _():
        pltpu.sync_copy(z1_hbm, tb_v)
        pltpu.sync_copy(tb_v, tacc_s.at[pl.ds(s * 1000, 1000)])

    pltpu.sync_copy(z2_hbm, zb_v)
    base = s * NBC * BR
    for q in range(NBC):
        pltpu.sync_copy(zb_v, acc_s.at[pl.ds(base + q * BR, BR)])

    @pl.when(s == 0)
    def _():
        pltpu.sync_copy(zb_v.at[pl.ds(0, TAIL)], acc_s.at[pl.ds(NN - TAIL, TAIL)])

    tile_w = c * NS + s

    def sidx(j):
        return sidx_v.at[lax.rem(lax.div(j, WIN), 2)].at[lax.rem(j, WIN)]

    def didx(j):
        return didx_v.at[lax.rem(lax.div(j, WIN), 2)].at[lax.rem(j, WIN)]

    pltpu.sync_copy(src3_hbm.at[tile_w].at[0], sidx_v.at[0])
    pltpu.sync_copy(dst3_hbm.at[tile_w].at[0], didx_v.at[0])
    plsc.subcore_barrier()

    def fire_g(j, p):
        pltpu.async_copy(hs_hbm.at[sidx(j)], rows_v.at[p], gsem)
        pltpu.async_copy(rin_hbm.at[didx(j)], rv_v.at[p], grsem)

    def wait_g(j, p):
        pltpu.make_async_copy(hs_hbm.at[sidx(j)], rows_v.at[p], gsem).wait()
        pltpu.make_async_copy(rin_hbm.at[didx(j)], rv_v.at[p], grsem).wait()

    def fire_s(j, p):
        pltpu.async_copy(rows_v.at[p], acc_s.at[didx(j)], ssem, add=True)
        pltpu.async_copy(rv_v.at[p], tacc_s.at[sidx(j)], stsem, add=True)

    def wait_s(j, p):
        pltpu.make_async_copy(rows_v.at[p], acc_s.at[didx(j)], ssem).wait()
        pltpu.make_async_copy(rv_v.at[p], tacc_s.at[sidx(j)], stsem).wait()

    def fire_idx(w, wp):
        pltpu.async_copy(src3_hbm.at[tile_w].at[w], sidx_v.at[wp], isem)
        pltpu.async_copy(dst3_hbm.at[tile_w].at[w], didx_v.at[wp], isem)

    def wait_idx(w, wp):
        pltpu.make_async_copy(src3_hbm.at[tile_w].at[w], sidx_v.at[wp],
                              isem).wait()
        pltpu.make_async_copy(dst3_hbm.at[tile_w].at[w], didx_v.at[wp],
                              isem).wait()

    fire_g(0, 0)

    def chunk(j, carry):
        p = lax.rem(j, 2)
        jw = lax.rem(j, WIN)
        w = lax.div(j, WIN)

        wait_g(j, p)

        @pl.when(j > 0)
        def _():
            wait_s(j - 1, 1 - p)

        @pl.when((jw == 12) & (w < ROWS_TILE_C // WIN - 1))
        def _():
            fire_idx(w + 1, lax.rem(w + 1, 2))

        @pl.when((jw == WIN - 1) & (w < ROWS_TILE_C // WIN - 1))
        def _():
            wait_idx(w + 1, lax.rem(w + 1, 2))

        @pl.when(j < ROWS_TILE_C - 1)
        def _():
            fire_g(j + 1, 1 - p)

        fire_s(j, p)
        return carry

    lax.fori_loop(0, ROWS_TILE_C, chunk, None)
    wait_s(ROWS_TILE_C - 1, (ROWS_TILE_C - 1) % 2)
    plsc.subcore_barrier()

    for q in range(NBC):
        pltpu.sync_copy(acc_s.at[pl.ds(base + q * BR, BR)], zb_v)
        pltpu.sync_copy(zb_v, agg_hbm.at[pl.ds(c * NN + base + q * BR, BR)])

    @pl.when(s == 0)
    def _():
        pltpu.sync_copy(acc_s.at[pl.ds(NN - TAIL, TAIL)],
                        zb_v.at[pl.ds(0, TAIL)])
        pltpu.sync_copy(zb_v.at[pl.ds(0, TAIL)],
                        agg_hbm.at[pl.ds(c * NN + NN - TAIL, TAIL)])

    @pl.when(s < 10)
    def _():
        pltpu.sync_copy(tacc_s.at[pl.ds(s * 1000, 1000)], tb_v)
        pltpu.sync_copy(tb_v, t_hbm.at[pl.ds(c * NN + s * 1000, 1000)])


def _scale_body(h_ref, do_ref, di_ref, hs_ref, rin_ref, rout_ref):
    ro = lax.rsqrt(jnp.maximum(do_ref[...], 1.0))
    ri = lax.rsqrt(jnp.maximum(di_ref[...], 1.0))
    hs_ref[...] = h_ref[...] * ro
    rin_ref[...] = ri
    rout_ref[...] = ro


_BB = 2000
_scale_call = pl.pallas_call(
    _scale_body,
    grid=(NN // _BB,),
    in_specs=[
        pl.BlockSpec((_BB, FIN), lambda i: (i, 0)),
        pl.BlockSpec((_BB, 1), lambda i: (i, 0)),
        pl.BlockSpec((_BB, 1), lambda i: (i, 0)),
    ],
    out_specs=[
        pl.BlockSpec((_BB, FIN), lambda i: (i, 0)),
        pl.BlockSpec((_BB, 1), lambda i: (i, 0)),
        pl.BlockSpec((_BB, 1), lambda i: (i, 0)),
    ],
    out_shape=[
        jax.ShapeDtypeStruct((NN, FIN), jnp.float32),
        jax.ShapeDtypeStruct((NN, 1), jnp.float32),
        jax.ShapeDtypeStruct((NN, 1), jnp.float32),
    ],
)


def _out_body(p_ref, t_ref, rin_ref, rout_ref, w1_ref, b1_ref, w2_ref, b2_ref,
              o_ref, v_acc):
    i = pl.program_id(0)

    @pl.when(i == 0)
    def _():
        v_acc[...] = jnp.zeros_like(v_acc)

    aggp = p_ref[0] + p_ref[1]
    x1 = jnp.dot(aggp, w1_ref[...], preferred_element_type=jnp.float32)
    x1 = jnp.maximum(x1 * rin_ref[...] + b1_ref[...], 0.0)
    w = rout_ref[...] * (t_ref[0] + t_ref[1])
    v_acc[...] += lax.dot_general(w, x1, (((0,), (0,)), ((), ())),
                                  preferred_element_type=jnp.float32)

    @pl.when(i == pl.num_programs(0) - 1)
    def _():
        o_ref[...] = (jnp.dot(v_acc[...], w2_ref[...],
                              preferred_element_type=jnp.float32) * (1.0 / NN)
                      + b2_ref[...])


_BD = 1000
_out_call = pl.pallas_call(
    _out_body,
    grid=(NN // _BD,),
    in_specs=[
        pl.BlockSpec((2, _BD, FIN), lambda i: (0, i, 0)),
        pl.BlockSpec((2, _BD, 1), lambda i: (0, i, 0)),
        pl.BlockSpec((_BD, 1), lambda i: (i, 0)),
        pl.BlockSpec((_BD, 1), lambda i: (i, 0)),
        pl.BlockSpec((FIN, FOUT), lambda i: (0, 0)),
        pl.BlockSpec((1, FOUT), lambda i: (0, 0)),
        pl.BlockSpec((FOUT, NCLS), lambda i: (0, 0)),
        pl.BlockSpec((1, NCLS), lambda i: (0, 0)),
    ],
    out_specs=pl.BlockSpec((1, NCLS), lambda i: (0, 0)),
    out_shape=jax.ShapeDtypeStruct((1, NCLS), jnp.float32),
    scratch_shapes=[pltpu.VMEM((1, FOUT), jnp.float32)],
)


def kernel(h, edge_index, W1, b1, W2, b2):
    ei = edge_index.astype(jnp.int32)
    src2 = ei[0].reshape(NC * NS, NWIN, WIN, K)
    dst2 = ei[1].reshape(NC * NS, NWIN, WIN, K)
    z1 = jnp.zeros((1000,), jnp.float32)
    z2 = jnp.zeros((BR, FIN), jnp.float32)

    sc = _get_sc_calls()
    deg = sc["deg"](src2, dst2, z1)
    do = deg[:NN].reshape(NN, 1)
    di = deg[NN:].reshape(NN, 1)

    hs, rin, rout = _scale_call(h, do, di)

    aggp, tp = sc["agg"](src2, dst2, hs, rin.reshape(NN), z2, z1)

    return _out_call(aggp.reshape(2, NN, FIN), tp.reshape(2, NN, 1),
                     rin, rout, W1, b1.reshape(1, FOUT), W2, b2.reshape(1, NCLS))


# speedup vs baseline: 15.9910x; 1.1724x over previous
import jax
import jax.numpy as jnp
from jax import lax
from jax.experimental import pallas as pl
from jax.experimental.pallas import tpu as pltpu
from jax.experimental.pallas import tpu_sc as plsc

NN = 10000
EE = 320000
FIN = 128
FOUT = 256
NCLS = 64

NC, NS = 2, 16
K = 80
WIN = 5
NBUF = 3
ROWS_ALL = EE // K
ROWS_TILE_A = ROWS_ALL // NS
ROWS_TILE_C = ROWS_ALL // (NC * NS)
NWIN = ROWS_TILE_C // WIN
RPT = NN // NS

def _deg_body(src3_hbm, dst3_hbm, z1_hbm, deg_hbm, idx_v, ones_v, tb_v, acc_s,
              dsem):
    c = lax.axis_index("c")
    s = lax.axis_index("s")

    @pl.when(s < 10)
    def _():
        pltpu.sync_copy(z1_hbm, tb_v)
        pltpu.sync_copy(tb_v, acc_s.at[pl.ds(s * 1000, 1000)])

    for u in range(K // 16):
        ones_v[pl.ds(u * 16, 16)] = jnp.full((16,), 1.0, jnp.float32)

    @pl.when(c == 0)
    def _():
        pltpu.sync_copy(src3_hbm.at[2 * s], idx_v.at[0])
        pltpu.sync_copy(src3_hbm.at[2 * s + 1], idx_v.at[1])

    @pl.when(c == 1)
    def _():
        pltpu.sync_copy(dst3_hbm.at[2 * s], idx_v.at[0])
        pltpu.sync_copy(dst3_hbm.at[2 * s + 1], idx_v.at[1])

    plsc.subcore_barrier()

    KF = 8

    def row(j):
        jj = lax.rem(j, NWIN * WIN)
        return (idx_v.at[lax.div(j, NWIN * WIN)]
                .at[lax.div(jj, WIN)].at[lax.rem(jj, WIN)])

    def chunk(j, carry):
        pltpu.async_copy(ones_v, acc_s.at[row(j)], dsem, add=True)

        @pl.when(j >= KF)
        def _():
            pltpu.make_async_copy(ones_v, acc_s.at[row(j)], dsem).wait()

        return carry

    lax.fori_loop(0, ROWS_TILE_A, chunk, None)
    for j in range(KF):
        pltpu.make_async_copy(ones_v, acc_s.at[row(j)], dsem).wait()
    plsc.subcore_barrier()

    @pl.when(s < 10)
    def _():
        pltpu.sync_copy(acc_s.at[pl.ds(s * 1000, 1000)], tb_v)
        pltpu.sync_copy(tb_v, deg_hbm.at[pl.ds(c * NN + s * 1000, 1000)])


_sc_calls = {}


def _get_sc_calls():
    if _sc_calls:
        return _sc_calls
    mesh = plsc.VectorSubcoreMesh(
        core_axis_name="c", subcore_axis_name="s",
        num_cores=NC, num_subcores=NS)
    _sc_calls["deg"] = pl.kernel(
        _deg_body,
        out_type=jax.ShapeDtypeStruct((2 * NN,), jnp.float32),
        mesh=mesh,
        scratch_types=[
            pltpu.VMEM((2, NWIN, WIN, K), jnp.int32),
            pltpu.VMEM((K,), jnp.float32),
            pltpu.VMEM((1000,), jnp.float32),
            pltpu.VMEM_SHARED((NN,), jnp.float32),
            pltpu.SemaphoreType.DMA,
        ],
    )
    _sc_calls["agg"] = pl.kernel(
        _agg_body,
        out_type=(jax.ShapeDtypeStruct((2 * NN, FIN), jnp.float32),
                  jax.ShapeDtypeStruct((2 * NN,), jnp.float32)),
        mesh=mesh,
        scratch_types=[
            pltpu.VMEM((2, WIN, K), jnp.int32),
            pltpu.VMEM((2, WIN, K), jnp.int32),
            pltpu.VMEM((NBUF, K, FIN), jnp.float32),
            pltpu.VMEM((NBUF, K), jnp.float32),
            pltpu.VMEM((BR, FIN), jnp.float32),
            pltpu.VMEM((1000,), jnp.float32),
            pltpu.VMEM_SHARED((NN, FIN), jnp.float32),
            pltpu.VMEM_SHARED((NN,), jnp.float32),
            pltpu.SemaphoreType.DMA((NBUF,)),
            pltpu.SemaphoreType.DMA((NBUF,)),
            pltpu.SemaphoreType.DMA((NBUF,)),
            pltpu.SemaphoreType.DMA((NBUF,)),
            pltpu.SemaphoreType.DMA,
        ],
    )
    return _sc_calls


BR = 16
NBC = 39
TAIL = NN - NS * NBC * BR


def _agg_body(src3_hbm, dst3_hbm, hs_hbm, rin_hbm, z2_hbm, z1_hbm,
              agg_hbm, t_hbm, sidx_v, didx_v, rows_v, rv_v, zb_v, tb_v,
              acc_s, tacc_s, gsem, grsem, ssem, stsem, isem):
    c = lax.axis_index("c")
    s = lax.axis_index("s")

    @pl.when(s < 10)
    def _():
        pltpu.sync_copy(z1_hbm, tb_v)
        pltpu.sync_copy(tb_v, tacc_s.at[pl.ds(s * 1000, 1000)])

    pltpu.sync_copy(z2_hbm, zb_v)
    base = s * NBC * BR
    for q in range(NBC):
        pltpu.sync_copy(zb_v, acc_s.at[pl.ds(base + q * BR, BR)])

    @pl.when(s == 0)
    def _():
        pltpu.sync_copy(zb_v.at[pl.ds(0, TAIL)], acc_s.at[pl.ds(NN - TAIL, TAIL)])

    tile_w = c * NS + s

    def sidx(j):
        return sidx_v.at[lax.rem(lax.div(j, WIN), 2)].at[lax.rem(j, WIN)]

    def didx(j):
        return didx_v.at[lax.rem(lax.div(j, WIN), 2)].at[lax.rem(j, WIN)]

    pltpu.sync_copy(src3_hbm.at[tile_w].at[0], sidx_v.at[0])
    pltpu.sync_copy(dst3_hbm.at[tile_w].at[0], didx_v.at[0])
    plsc.subcore_barrier()

    def fire_g(j, p):
        pltpu.async_copy(hs_hbm.at[sidx(j)], rows_v.at[p], gsem.at[p])
        pltpu.async_copy(rin_hbm.at[didx(j)], rv_v.at[p], grsem.at[p])

    def wait_g(j, p):
        pltpu.make_async_copy(hs_hbm.at[sidx(j)], rows_v.at[p],
                              gsem.at[p]).wait()
        pltpu.make_async_copy(rin_hbm.at[didx(j)], rv_v.at[p],
                              grsem.at[p]).wait()

    def fire_s(j, p):
        pltpu.async_copy(rows_v.at[p], acc_s.at[didx(j)], ssem.at[p], add=True)
        pltpu.async_copy(rv_v.at[p], tacc_s.at[sidx(j)], stsem.at[p], add=True)

    def wait_s(j, p):
        pltpu.make_async_copy(rows_v.at[p], acc_s.at[didx(j)],
                              ssem.at[p]).wait()
        pltpu.make_async_copy(rv_v.at[p], tacc_s.at[sidx(j)],
                              stsem.at[p]).wait()

    def fire_idx(w, wp):
        pltpu.async_copy(src3_hbm.at[tile_w].at[w], sidx_v.at[wp], isem)
        pltpu.async_copy(dst3_hbm.at[tile_w].at[w], didx_v.at[wp], isem)

    def wait_idx(w, wp):
        pltpu.make_async_copy(src3_hbm.at[tile_w].at[w], sidx_v.at[wp],
                              isem).wait()
        pltpu.make_async_copy(dst3_hbm.at[tile_w].at[w], didx_v.at[wp],
                              isem).wait()

    fire_g(0, 0)
    fire_g(1, 1)

    def chunk(j, carry):
        p = lax.rem(j, NBUF)
        jw = lax.rem(j, WIN)
        w = lax.div(j, WIN)

        wait_g(j, p)

        @pl.when(j > 0)
        def _():
            wait_s(j - 1, lax.rem(j - 1, NBUF))

        @pl.when((jw == 0) & (w < NWIN - 1))
        def _():
            fire_idx(w + 1, lax.rem(w + 1, 2))

        @pl.when((jw == WIN - 3) & (w < NWIN - 1))
        def _():
            wait_idx(w + 1, lax.rem(w + 1, 2))

        @pl.when(j < ROWS_TILE_C - 2)
        def _():
            fire_g(j + 2, lax.rem(j + 2, NBUF))

        fire_s(j, p)
        return carry

    lax.fori_loop(0, ROWS_TILE_C, chunk, None)
    wait_s(ROWS_TILE_C - 1, (ROWS_TILE_C - 1) % NBUF)
    plsc.subcore_barrier()

    for q in range(NBC):
        pltpu.sync_copy(acc_s.at[pl.ds(base + q * BR, BR)], zb_v)
        pltpu.sync_copy(zb_v, agg_hbm.at[pl.ds(c * NN + base + q * BR, BR)])

    @pl.when(s == 0)
    def _():
        pltpu.sync_copy(acc_s.at[pl.ds(NN - TAIL, TAIL)],
                        zb_v.at[pl.ds(0, TAIL)])
        pltpu.sync_copy(zb_v.at[pl.ds(0, TAIL)],
                        agg_hbm.at[pl.ds(c * NN + NN - TAIL, TAIL)])

    @pl.when(s < 10)
    def _():
        pltpu.sync_copy(tacc_s.at[pl.ds(s * 1000, 1000)], tb_v)
        pltpu.sync_copy(tb_v, t_hbm.at[pl.ds(c * NN + s * 1000, 1000)])


def _scale_body(h_ref, do_ref, di_ref, hs_ref, rin_ref, rout_ref):
    ro = lax.rsqrt(jnp.maximum(do_ref[...], 1.0))
    ri = lax.rsqrt(jnp.maximum(di_ref[...], 1.0))
    hs_ref[...] = h_ref[...] * ro
    rin_ref[...] = ri
    rout_ref[...] = ro


_BB = 2000
_scale_call = pl.pallas_call(
    _scale_body,
    grid=(NN // _BB,),
    in_specs=[
        pl.BlockSpec((_BB, FIN), lambda i: (i, 0)),
        pl.BlockSpec((_BB, 1), lambda i: (i, 0)),
        pl.BlockSpec((_BB, 1), lambda i: (i, 0)),
    ],
    out_specs=[
        pl.BlockSpec((_BB, FIN), lambda i: (i, 0)),
        pl.BlockSpec((_BB, 1), lambda i: (i, 0)),
        pl.BlockSpec((_BB, 1), lambda i: (i, 0)),
    ],
    out_shape=[
        jax.ShapeDtypeStruct((NN, FIN), jnp.float32),
        jax.ShapeDtypeStruct((NN, 1), jnp.float32),
        jax.ShapeDtypeStruct((NN, 1), jnp.float32),
    ],
)


def _out_body(p_ref, t_ref, rin_ref, rout_ref, w1_ref, b1_ref, w2_ref, b2_ref,
              o_ref, v_acc):
    i = pl.program_id(0)

    @pl.when(i == 0)
    def _():
        v_acc[...] = jnp.zeros_like(v_acc)

    aggp = p_ref[0] + p_ref[1]
    x1 = jnp.dot(aggp, w1_ref[...], preferred_element_type=jnp.float32)
    x1 = jnp.maximum(x1 * rin_ref[...] + b1_ref[...], 0.0)
    w = rout_ref[...] * (t_ref[0] + t_ref[1])
    v_acc[...] += lax.dot_general(w, x1, (((0,), (0,)), ((), ())),
                                  preferred_element_type=jnp.float32)

    @pl.when(i == pl.num_programs(0) - 1)
    def _():
        o_ref[...] = (jnp.dot(v_acc[...], w2_ref[...],
                              preferred_element_type=jnp.float32) * (1.0 / NN)
                      + b2_ref[...])


_BD = 1000
_out_call = pl.pallas_call(
    _out_body,
    grid=(NN // _BD,),
    in_specs=[
        pl.BlockSpec((2, _BD, FIN), lambda i: (0, i, 0)),
        pl.BlockSpec((2, _BD, 1), lambda i: (0, i, 0)),
        pl.BlockSpec((_BD, 1), lambda i: (i, 0)),
        pl.BlockSpec((_BD, 1), lambda i: (i, 0)),
        pl.BlockSpec((FIN, FOUT), lambda i: (0, 0)),
        pl.BlockSpec((1, FOUT), lambda i: (0, 0)),
        pl.BlockSpec((FOUT, NCLS), lambda i: (0, 0)),
        pl.BlockSpec((1, NCLS), lambda i: (0, 0)),
    ],
    out_specs=pl.BlockSpec((1, NCLS), lambda i: (0, 0)),
    out_shape=jax.ShapeDtypeStruct((1, NCLS), jnp.float32),
    scratch_shapes=[pltpu.VMEM((1, FOUT), jnp.float32)],
)


def kernel(h, edge_index, W1, b1, W2, b2):
    ei = edge_index.astype(jnp.int32)
    src2 = ei[0].reshape(NC * NS, NWIN, WIN, K)
    dst2 = ei[1].reshape(NC * NS, NWIN, WIN, K)
    z1 = jnp.zeros((1000,), jnp.float32)
    z2 = jnp.zeros((BR, FIN), jnp.float32)

    sc = _get_sc_calls()
    deg = sc["deg"](src2, dst2, z1)
    do = deg[:NN].reshape(NN, 1)
    di = deg[NN:].reshape(NN, 1)

    hs, rin, rout = _scale_call(h, do, di)

    aggp, tp = sc["agg"](src2, dst2, hs, rin.reshape(NN), z2, z1)

    return _out_call(aggp.reshape(2, NN, FIN), tp.reshape(2, NN, 1),
                     rin, rout, W1, b1.reshape(1, FOUT), W2, b2.reshape(1, NCLS))


# speedup vs baseline: 17.3599x; 1.0856x over previous
import jax
import jax.numpy as jnp
from jax import lax
from jax.experimental import pallas as pl
from jax.experimental.pallas import tpu as pltpu
from jax.experimental.pallas import tpu_sc as plsc

NN = 10000
EE = 320000
FIN = 128
FOUT = 256
NCLS = 64

NC, NS = 2, 16
K = 80
WIN = 5
NBUF = 3
ROWS_ALL = EE // K
ROWS_TILE_A = ROWS_ALL // NS
ROWS_TILE_C = ROWS_ALL // (NC * NS)
NWIN = ROWS_TILE_C // WIN
RPT = NN // NS

def _deg_body(src3_hbm, dst3_hbm, z1_hbm, deg_hbm, idx_v, ones_v, tb_v, acc_s,
              dsem):
    c = lax.axis_index("c")
    s = lax.axis_index("s")

    @pl.when(s < 10)
    def _():
        pltpu.sync_copy(z1_hbm, tb_v)
        pltpu.sync_copy(tb_v, acc_s.at[pl.ds(s * 1000, 1000)])

    for u in range(K // 16):
        ones_v[pl.ds(u * 16, 16)] = jnp.full((16,), 1.0, jnp.float32)

    @pl.when(c == 0)
    def _():
        pltpu.sync_copy(src3_hbm.at[2 * s], idx_v.at[0])
        pltpu.sync_copy(src3_hbm.at[2 * s + 1], idx_v.at[1])

    @pl.when(c == 1)
    def _():
        pltpu.sync_copy(dst3_hbm.at[2 * s], idx_v.at[0])
        pltpu.sync_copy(dst3_hbm.at[2 * s + 1], idx_v.at[1])

    plsc.subcore_barrier()

    KF = 8

    def row(j):
        jj = lax.rem(j, NWIN * WIN)
        return (idx_v.at[lax.div(j, NWIN * WIN)]
                .at[lax.div(jj, WIN)].at[lax.rem(jj, WIN)])

    def chunk(j, carry):
        pltpu.async_copy(ones_v, acc_s.at[row(j)], dsem, add=True)

        @pl.when(j >= KF)
        def _():
            pltpu.make_async_copy(ones_v, acc_s.at[row(j)], dsem).wait()

        return carry

    lax.fori_loop(0, ROWS_TILE_A, chunk, None)
    for j in range(KF):
        pltpu.make_async_copy(ones_v, acc_s.at[row(j)], dsem).wait()
    plsc.subcore_barrier()

    @pl.when(s < 10)
    def _():
        pltpu.sync_copy(acc_s.at[pl.ds(s * 1000, 1000)], tb_v)
        pltpu.sync_copy(tb_v, deg_hbm.at[pl.ds(c * NN + s * 1000, 1000)])


_sc_calls = {}


def _get_sc_calls():
    if _sc_calls:
        return _sc_calls
    mesh = plsc.VectorSubcoreMesh(
        core_axis_name="c", subcore_axis_name="s",
        num_cores=NC, num_subcores=NS)
    _sc_calls["deg"] = pl.kernel(
        _deg_body,
        out_type=jax.ShapeDtypeStruct((2 * NN,), jnp.float32),
        mesh=mesh,
        scratch_types=[
            pltpu.VMEM((2, NWIN, WIN, K), jnp.int32),
            pltpu.VMEM((K,), jnp.float32),
            pltpu.VMEM((1000,), jnp.float32),
            pltpu.VMEM_SHARED((NN,), jnp.float32),
            pltpu.SemaphoreType.DMA,
        ],
    )
    _sc_calls["agg"] = pl.kernel(
        _agg_body,
        out_type=(jax.ShapeDtypeStruct((2 * NN, FIN), jnp.float32),
                  jax.ShapeDtypeStruct((2 * NN,), jnp.float32)),
        mesh=mesh,
        scratch_types=[
            pltpu.VMEM((2, WIN, K), jnp.int32),
            pltpu.VMEM((2, WIN, K), jnp.int32),
            pltpu.VMEM((NBUF, K, FIN), jnp.float32),
            pltpu.VMEM((NBUF, K), jnp.float32),
            pltpu.VMEM((BR, FIN), jnp.float32),
            pltpu.VMEM((1000,), jnp.float32),
            pltpu.VMEM_SHARED((NN, FIN), jnp.float32),
            pltpu.VMEM_SHARED((NN,), jnp.float32),
            pltpu.VMEM_SHARED((NN,), jnp.float32),
            pltpu.SemaphoreType.DMA((NBUF,)),
            pltpu.SemaphoreType.DMA((NBUF,)),
            pltpu.SemaphoreType.DMA((NBUF,)),
            pltpu.SemaphoreType.DMA((NBUF,)),
            pltpu.SemaphoreType.DMA,
        ],
    )
    return _sc_calls


BR = 16
NBC = 39
TAIL = NN - NS * NBC * BR


def _agg_body(src3_hbm, dst3_hbm, hs_hbm, rin_hbm, z2_hbm, z1_hbm,
              agg_hbm, t_hbm, sidx_v, didx_v, rows_v, rv_v, zb_v, tb_v,
              acc_s, tacc_s, rin_s, gsem, grsem, ssem, stsem, isem):
    c = lax.axis_index("c")
    s = lax.axis_index("s")

    @pl.when(s < 10)
    def _():
        pltpu.sync_copy(z1_hbm, tb_v)
        pltpu.sync_copy(tb_v, tacc_s.at[pl.ds(s * 1000, 1000)])
        pltpu.sync_copy(rin_hbm.at[pl.ds(s * 1000, 1000)], tb_v)
        pltpu.sync_copy(tb_v, rin_s.at[pl.ds(s * 1000, 1000)])

    pltpu.sync_copy(z2_hbm, zb_v)
    base = s * NBC * BR
    for q in range(NBC):
        pltpu.sync_copy(zb_v, acc_s.at[pl.ds(base + q * BR, BR)])

    @pl.when(s == 0)
    def _():
        pltpu.sync_copy(zb_v.at[pl.ds(0, TAIL)], acc_s.at[pl.ds(NN - TAIL, TAIL)])

    tile_w = c * NS + s

    def sidx(j):
        return sidx_v.at[lax.rem(lax.div(j, WIN), 2)].at[lax.rem(j, WIN)]

    def didx(j):
        return didx_v.at[lax.rem(lax.div(j, WIN), 2)].at[lax.rem(j, WIN)]

    pltpu.sync_copy(src3_hbm.at[tile_w].at[0], sidx_v.at[0])
    pltpu.sync_copy(dst3_hbm.at[tile_w].at[0], didx_v.at[0])
    plsc.subcore_barrier()

    def fire_g(j, p):
        pltpu.async_copy(hs_hbm.at[sidx(j)], rows_v.at[p], gsem.at[p])
        pltpu.async_copy(rin_s.at[didx(j)], rv_v.at[p], grsem.at[p])

    def wait_g(j, p):
        pltpu.make_async_copy(hs_hbm.at[sidx(j)], rows_v.at[p],
                              gsem.at[p]).wait()
        pltpu.make_async_copy(rin_s.at[didx(j)], rv_v.at[p],
                              grsem.at[p]).wait()

    def fire_s(j, p):
        pltpu.async_copy(rows_v.at[p], acc_s.at[didx(j)], ssem.at[p], add=True)
        pltpu.async_copy(rv_v.at[p], tacc_s.at[sidx(j)], stsem.at[p], add=True)

    def wait_s(j, p):
        pltpu.make_async_copy(rows_v.at[p], acc_s.at[didx(j)],
                              ssem.at[p]).wait()
        pltpu.make_async_copy(rv_v.at[p], tacc_s.at[sidx(j)],
                              stsem.at[p]).wait()

    def fire_idx(w, wp):
        pltpu.async_copy(src3_hbm.at[tile_w].at[w], sidx_v.at[wp], isem)
        pltpu.async_copy(dst3_hbm.at[tile_w].at[w], didx_v.at[wp], isem)

    def wait_idx(w, wp):
        pltpu.make_async_copy(src3_hbm.at[tile_w].at[w], sidx_v.at[wp],
                              isem).wait()
        pltpu.make_async_copy(dst3_hbm.at[tile_w].at[w], didx_v.at[wp],
                              isem).wait()

    fire_g(0, 0)
    fire_g(1, 1)

    def chunk(j, carry):
        p = lax.rem(j, NBUF)
        jw = lax.rem(j, WIN)
        w = lax.div(j, WIN)

        wait_g(j, p)

        @pl.when(j > 0)
        def _():
            wait_s(j - 1, lax.rem(j - 1, NBUF))

        @pl.when((jw == 0) & (w < NWIN - 1))
        def _():
            fire_idx(w + 1, lax.rem(w + 1, 2))

        @pl.when((jw == WIN - 3) & (w < NWIN - 1))
        def _():
            wait_idx(w + 1, lax.rem(w + 1, 2))

        @pl.when(j < ROWS_TILE_C - 2)
        def _():
            fire_g(j + 2, lax.rem(j + 2, NBUF))

        fire_s(j, p)
        return carry

    lax.fori_loop(0, ROWS_TILE_C, chunk, None)
    wait_s(ROWS_TILE_C - 1, (ROWS_TILE_C - 1) % NBUF)
    plsc.subcore_barrier()

    for q in range(NBC):
        pltpu.sync_copy(acc_s.at[pl.ds(base + q * BR, BR)], zb_v)
        pltpu.sync_copy(zb_v, agg_hbm.at[pl.ds(c * NN + base + q * BR, BR)])

    @pl.when(s == 0)
    def _():
        pltpu.sync_copy(acc_s.at[pl.ds(NN - TAIL, TAIL)],
                        zb_v.at[pl.ds(0, TAIL)])
        pltpu.sync_copy(zb_v.at[pl.ds(0, TAIL)],
                        agg_hbm.at[pl.ds(c * NN + NN - TAIL, TAIL)])

    @pl.when(s < 10)
    def _():
        pltpu.sync_copy(tacc_s.at[pl.ds(s * 1000, 1000)], tb_v)
        pltpu.sync_copy(tb_v, t_hbm.at[pl.ds(c * NN + s * 1000, 1000)])


def _scale_body(h_ref, do_ref, di_ref, hs_ref, rin_ref, rout_ref):
    ro = lax.rsqrt(jnp.maximum(do_ref[...], 1.0))
    ri = lax.rsqrt(jnp.maximum(di_ref[...], 1.0))
    hs_ref[...] = h_ref[...] * ro
    rin_ref[...] = ri
    rout_ref[...] = ro


_BB = 2000
_scale_call = pl.pallas_call(
    _scale_body,
    grid=(NN // _BB,),
    in_specs=[
        pl.BlockSpec((_BB, FIN), lambda i: (i, 0)),
        pl.BlockSpec((_BB, 1), lambda i: (i, 0)),
        pl.BlockSpec((_BB, 1), lambda i: (i, 0)),
    ],
    out_specs=[
        pl.BlockSpec((_BB, FIN), lambda i: (i, 0)),
        pl.BlockSpec((_BB, 1), lambda i: (i, 0)),
        pl.BlockSpec((_BB, 1), lambda i: (i, 0)),
    ],
    out_shape=[
        jax.ShapeDtypeStruct((NN, FIN), jnp.float32),
        jax.ShapeDtypeStruct((NN, 1), jnp.float32),
        jax.ShapeDtypeStruct((NN, 1), jnp.float32),
    ],
)


def _out_body(p_ref, t_ref, rin_ref, rout_ref, w1_ref, b1_ref, w2_ref, b2_ref,
              o_ref, v_acc):
    i = pl.program_id(0)

    @pl.when(i == 0)
    def _():
        v_acc[...] = jnp.zeros_like(v_acc)

    aggp = p_ref[0] + p_ref[1]
    x1 = jnp.dot(aggp, w1_ref[...], preferred_element_type=jnp.float32)
    x1 = jnp.maximum(x1 * rin_ref[...] + b1_ref[...], 0.0)
    w = rout_ref[...] * (t_ref[0] + t_ref[1])
    v_acc[...] += lax.dot_general(w, x1, (((0,), (0,)), ((), ())),
                                  preferred_element_type=jnp.float32)

    @pl.when(i == pl.num_programs(0) - 1)
    def _():
        o_ref[...] = (jnp.dot(v_acc[...], w2_ref[...],
                              preferred_element_type=jnp.float32) * (1.0 / NN)
                      + b2_ref[...])


_BD = 1000
_out_call = pl.pallas_call(
    _out_body,
    grid=(NN // _BD,),
    in_specs=[
        pl.BlockSpec((2, _BD, FIN), lambda i: (0, i, 0)),
        pl.BlockSpec((2, _BD, 1), lambda i: (0, i, 0)),
        pl.BlockSpec((_BD, 1), lambda i: (i, 0)),
        pl.BlockSpec((_BD, 1), lambda i: (i, 0)),
        pl.BlockSpec((FIN, FOUT), lambda i: (0, 0)),
        pl.BlockSpec((1, FOUT), lambda i: (0, 0)),
        pl.BlockSpec((FOUT, NCLS), lambda i: (0, 0)),
        pl.BlockSpec((1, NCLS), lambda i: (0, 0)),
    ],
    out_specs=pl.BlockSpec((1, NCLS), lambda i: (0, 0)),
    out_shape=jax.ShapeDtypeStruct((1, NCLS), jnp.float32),
    scratch_shapes=[pltpu.VMEM((1, FOUT), jnp.float32)],
)


def kernel(h, edge_index, W1, b1, W2, b2):
    ei = edge_index.astype(jnp.int32)
    src2 = ei[0].reshape(NC * NS, NWIN, WIN, K)
    dst2 = ei[1].reshape(NC * NS, NWIN, WIN, K)
    z1 = jnp.zeros((1000,), jnp.float32)
    z2 = jnp.zeros((BR, FIN), jnp.float32)

    sc = _get_sc_calls()
    deg = sc["deg"](src2, dst2, z1)
    do = deg[:NN].reshape(NN, 1)
    di = deg[NN:].reshape(NN, 1)

    hs, rin, rout = _scale_call(h, do, di)

    aggp, tp = sc["agg"](src2, dst2, hs, rin.reshape(NN), z2, z1)

    return _out_call(aggp.reshape(2, NN, FIN), tp.reshape(2, NN, 1),
                     rin, rout, W1, b1.reshape(1, FOUT), W2, b2.reshape(1, NCLS))


# speedup vs baseline: 20.5948x; 1.1863x over previous
import jax
import jax.numpy as jnp
from jax import lax
from jax.experimental import pallas as pl
from jax.experimental.pallas import tpu as pltpu
from jax.experimental.pallas import tpu_sc as plsc

NN = 10000
EE = 320000
FIN = 128
FOUT = 256
NCLS = 64

NC, NS = 2, 16
K = 80
WIN = 5
NBUF = 3
ROWS_ALL = EE // K
ROWS_TILE_A = ROWS_ALL // NS
ROWS_TILE_C = ROWS_ALL // (NC * NS)
NWIN = ROWS_TILE_C // WIN
RPT = NN // NS

def _deg_body(src3_hbm, dst3_hbm, z1_hbm, deg_hbm, idx_v, ones_v, tb_v, acc_s,
              dsem):
    c = lax.axis_index("c")
    s = lax.axis_index("s")

    @pl.when(s < 10)
    def _():
        pltpu.sync_copy(z1_hbm, tb_v)
        pltpu.sync_copy(tb_v, acc_s.at[pl.ds(s * 1000, 1000)])

    for u in range(K // 16):
        ones_v[pl.ds(u * 16, 16)] = jnp.full((16,), 1.0, jnp.float32)

    @pl.when(c == 0)
    def _():
        pltpu.sync_copy(src3_hbm.at[2 * s], idx_v.at[0])
        pltpu.sync_copy(src3_hbm.at[2 * s + 1], idx_v.at[1])

    @pl.when(c == 1)
    def _():
        pltpu.sync_copy(dst3_hbm.at[2 * s], idx_v.at[0])
        pltpu.sync_copy(dst3_hbm.at[2 * s + 1], idx_v.at[1])

    plsc.subcore_barrier()

    KF = 8

    def row(j):
        jj = lax.rem(j, NWIN * WIN)
        return (idx_v.at[lax.div(j, NWIN * WIN)]
                .at[lax.div(jj, WIN)].at[lax.rem(jj, WIN)])

    def chunk(j, carry):
        pltpu.async_copy(ones_v, acc_s.at[row(j)], dsem, add=True)

        @pl.when(j >= KF)
        def _():
            pltpu.make_async_copy(ones_v, acc_s.at[row(j)], dsem).wait()

        return carry

    lax.fori_loop(0, ROWS_TILE_A, chunk, None)
    for j in range(KF):
        pltpu.make_async_copy(ones_v, acc_s.at[row(j)], dsem).wait()
    plsc.subcore_barrier()

    @pl.when(s < 10)
    def _():
        pltpu.sync_copy(acc_s.at[pl.ds(s * 1000, 1000)], tb_v)
        pltpu.sync_copy(tb_v, deg_hbm.at[pl.ds(c * NN + s * 1000, 1000)])


_sc_calls = {}


def _get_sc_calls():
    if _sc_calls:
        return _sc_calls
    mesh = plsc.VectorSubcoreMesh(
        core_axis_name="c", subcore_axis_name="s",
        num_cores=NC, num_subcores=NS)
    _sc_calls["deg"] = pl.kernel(
        _deg_body,
        out_type=jax.ShapeDtypeStruct((2 * NN,), jnp.float32),
        mesh=mesh,
        scratch_types=[
            pltpu.VMEM((2, NWIN, WIN, K), jnp.int32),
            pltpu.VMEM((K,), jnp.float32),
            pltpu.VMEM((1000,), jnp.float32),
            pltpu.VMEM_SHARED((NN,), jnp.float32),
            pltpu.SemaphoreType.DMA,
        ],
    )
    _sc_calls["agg"] = pl.kernel(
        _agg_body,
        out_type=(jax.ShapeDtypeStruct((2 * NN, FIN), jnp.float32),
                  jax.ShapeDtypeStruct((2 * NN,), jnp.float32)),
        mesh=mesh,
        scratch_types=[
            pltpu.VMEM((2, WIN, K), jnp.int32),
            pltpu.VMEM((2, WIN, K), jnp.int32),
            pltpu.VMEM((NBUF, K, FIN), jnp.float32),
            pltpu.VMEM((NBUF, K), jnp.float32),
            pltpu.VMEM((BR, FIN), jnp.float32),
            pltpu.VMEM((1000,), jnp.float32),
            pltpu.VMEM_SHARED((NN, FIN), jnp.float32),
            pltpu.VMEM_SHARED((NN,), jnp.float32),
            pltpu.VMEM_SHARED((NN,), jnp.float32),
            pltpu.SemaphoreType.DMA((NBUF,)),
            pltpu.SemaphoreType.DMA((NBUF,)),
            pltpu.SemaphoreType.DMA((NBUF,)),
            pltpu.SemaphoreType.DMA((NBUF,)),
            pltpu.SemaphoreType.DMA,
        ],
    )
    return _sc_calls


BR = 16
NBC = 39
TAIL = NN - NS * NBC * BR


def _agg_body(src3_hbm, dst3_hbm, hs_hbm, rin_hbm, z2_hbm, z1_hbm,
              agg_hbm, t_hbm, sidx_v, didx_v, rows_v, rv_v, zb_v, tb_v,
              acc_s, tacc_s, rin_s, gsem, grsem, ssem, stsem, isem):
    c = lax.axis_index("c")
    s = lax.axis_index("s")

    @pl.when(s < 10)
    def _():
        pltpu.sync_copy(z1_hbm, tb_v)
        pltpu.sync_copy(tb_v, tacc_s.at[pl.ds(s * 1000, 1000)])
        pltpu.sync_copy(rin_hbm.at[pl.ds(s * 1000, 1000)], tb_v)
        pltpu.sync_copy(tb_v, rin_s.at[pl.ds(s * 1000, 1000)])

    pltpu.sync_copy(z2_hbm, zb_v)
    base = s * NBC * BR
    for q in range(NBC):
        pltpu.sync_copy(zb_v, acc_s.at[pl.ds(base + q * BR, BR)])

    @pl.when(s == 0)
    def _():
        pltpu.sync_copy(zb_v.at[pl.ds(0, TAIL)], acc_s.at[pl.ds(NN - TAIL, TAIL)])

    tile_w = c * NS + s

    def sidx(j):
        return sidx_v.at[lax.rem(lax.div(j, WIN), 2)].at[lax.rem(j, WIN)]

    def didx(j):
        return didx_v.at[lax.rem(lax.div(j, WIN), 2)].at[lax.rem(j, WIN)]

    pltpu.sync_copy(src3_hbm.at[tile_w].at[0], sidx_v.at[0])
    pltpu.sync_copy(dst3_hbm.at[tile_w].at[0], didx_v.at[0])
    plsc.subcore_barrier()

    def fire_g(j, p):
        pltpu.async_copy(hs_hbm.at[sidx(j)], rows_v.at[p], gsem.at[p])
        pltpu.async_copy(rin_s.at[didx(j)], rv_v.at[p], grsem.at[p])

    def wait_g(j, p):
        pltpu.make_async_copy(hs_hbm.at[sidx(j)], rows_v.at[p],
                              gsem.at[p]).wait()
        pltpu.make_async_copy(rin_s.at[didx(j)], rv_v.at[p],
                              grsem.at[p]).wait()

    def fire_s(j, p):
        pltpu.async_copy(rows_v.at[p], acc_s.at[didx(j)], ssem.at[p], add=True)
        pltpu.async_copy(rv_v.at[p], tacc_s.at[sidx(j)], stsem.at[p], add=True)

    def wait_s(j, p):
        pltpu.make_async_copy(rows_v.at[p], acc_s.at[didx(j)],
                              ssem.at[p]).wait()
        pltpu.make_async_copy(rv_v.at[p], tacc_s.at[sidx(j)],
                              stsem.at[p]).wait()

    def fire_idx(w, wp):
        pltpu.async_copy(src3_hbm.at[tile_w].at[w], sidx_v.at[wp], isem)
        pltpu.async_copy(dst3_hbm.at[tile_w].at[w], didx_v.at[wp], isem)

    def wait_idx(w, wp):
        pltpu.make_async_copy(src3_hbm.at[tile_w].at[w], sidx_v.at[wp],
                              isem).wait()
        pltpu.make_async_copy(dst3_hbm.at[tile_w].at[w], didx_v.at[wp],
                              isem).wait()

    fire_g(0, 0)
    fire_g(1, 1)

    def chunk(j, carry):
        p = lax.rem(j, NBUF)
        jw = lax.rem(j, WIN)
        w = lax.div(j, WIN)

        wait_g(j, p)

        @pl.when(j > 0)
        def _():
            wait_s(j - 1, lax.rem(j - 1, NBUF))

        @pl.when((jw == 0) & (w < NWIN - 1))
        def _():
            fire_idx(w + 1, lax.rem(w + 1, 2))

        @pl.when((jw == WIN - 3) & (w < NWIN - 1))
        def _():
            wait_idx(w + 1, lax.rem(w + 1, 2))

        @pl.when(j < ROWS_TILE_C - 2)
        def _():
            fire_g(j + 2, lax.rem(j + 2, NBUF))

        fire_s(j, p)
        return carry

    lax.fori_loop(0, ROWS_TILE_C, chunk, None)
    wait_s(ROWS_TILE_C - 1, (ROWS_TILE_C - 1) % NBUF)
    plsc.subcore_barrier()

    for q in range(NBC):
        pltpu.sync_copy(acc_s.at[pl.ds(base + q * BR, BR)], zb_v)
        pltpu.sync_copy(zb_v, agg_hbm.at[pl.ds(c * NN + base + q * BR, BR)])

    @pl.when(s == 0)
    def _():
        pltpu.sync_copy(acc_s.at[pl.ds(NN - TAIL, TAIL)],
                        zb_v.at[pl.ds(0, TAIL)])
        pltpu.sync_copy(zb_v.at[pl.ds(0, TAIL)],
                        agg_hbm.at[pl.ds(c * NN + NN - TAIL, TAIL)])

    @pl.when(s < 10)
    def _():
        pltpu.sync_copy(tacc_s.at[pl.ds(s * 1000, 1000)], tb_v)
        pltpu.sync_copy(tb_v, t_hbm.at[pl.ds(c * NN + s * 1000, 1000)])


def _scale_body(h_ref, deg_ref, hs_ref, rio_ref, rib_ref):
    rr = lax.rsqrt(jnp.maximum(deg_ref[...], 1.0))
    rt = jnp.transpose(rr)
    hs_ref[...] = h_ref[...] * rt[:, 0:1]
    rio_ref[...] = rr
    rib_ref[...] = jnp.broadcast_to(rt[:, 1:2], (NN, FIN))


_scale_call = pl.pallas_call(
    _scale_body,
    out_shape=[
        jax.ShapeDtypeStruct((NN, FIN), jnp.float32),
        jax.ShapeDtypeStruct((2, NN), jnp.float32),
        jax.ShapeDtypeStruct((NN, FIN), jnp.float32),
    ],
)


def _out_body(p_ref, t_ref, rio_ref, rib_ref, w1_ref, b1_ref, w2_ref, b2_ref,
              o_ref):
    aggp = p_ref[pl.ds(0, NN), :] + p_ref[pl.ds(NN, NN), :]
    x1 = jnp.dot(aggp * rib_ref[...], w1_ref[...],
                 preferred_element_type=jnp.float32)
    x1 = jnp.maximum(x1 + b1_ref[...], 0.0)
    w = rio_ref[0:1, :] * (t_ref[0:1, :] + t_ref[1:2, :])
    v = lax.dot_general(w, x1, (((1,), (0,)), ((), ())),
                        preferred_element_type=jnp.float32)
    o_ref[...] = (jnp.dot(v, w2_ref[...],
                          preferred_element_type=jnp.float32) * (1.0 / NN)
                  + b2_ref[...])


_out_call = pl.pallas_call(
    _out_body,
    out_shape=jax.ShapeDtypeStruct((1, NCLS), jnp.float32),
)


def kernel(h, edge_index, W1, b1, W2, b2):
    ei = edge_index.astype(jnp.int32)
    src2 = ei[0].reshape(NC * NS, NWIN, WIN, K)
    dst2 = ei[1].reshape(NC * NS, NWIN, WIN, K)
    z1 = jnp.zeros((1000,), jnp.float32)
    z2 = jnp.zeros((BR, FIN), jnp.float32)

    sc = _get_sc_calls()
    deg = sc["deg"](src2, dst2, z1)

    hs, rio, rib = _scale_call(h, deg.reshape(2, NN))

    aggp, tp = sc["agg"](src2, dst2, hs, rio[1], z2, z1)

    return _out_call(aggp, tp.reshape(2, NN), rio, rib,
                     W1, b1.reshape(1, FOUT), W2, b2.reshape(1, NCLS))
